# baseline (speedup 1.0000x reference)
import functools

import jax
import jax.numpy as jnp
from jax import lax
from jax.experimental import pallas as pl
from jax.experimental.pallas import tpu as pltpu
from jax.experimental.pallas import tpu_sc as plsc

NUM_CORES = 2
NUM_SUBCORES = 16
NUM_WORKERS = NUM_CORES * NUM_SUBCORES

BATCH = 16384
SEQ = 50
EMBED_DIM = 32
VOCAB = 1000000

PACKED_ROWS = VOCAB // 4
NBLK_FULL = VOCAB // 128
TAIL_COLS = VOCAB - NBLK_FULL * 128
PACK_ITERS = (NBLK_FULL + NUM_WORKERS - 1) // NUM_WORKERS

JBLKS = BATCH // 128
J_PER_W = JBLKS // NUM_WORKERS
UNITS = SEQ * J_PER_W
UPAIRS = UNITS // 2


def _worker_id():
  return lax.axis_index("s") * NUM_CORES + lax.axis_index("c")


def _iota16():
  return lax.iota(jnp.int32, 16)


def _pack_table(table_t, tail_pk):
  mesh = plsc.VectorSubcoreMesh(core_axis_name="c", subcore_axis_name="s")

  @functools.partial(
      pl.kernel,
      out_type=jax.ShapeDtypeStruct((PACKED_ROWS, 128), jnp.float32),
      mesh=mesh,
      scratch_types=[
          [pltpu.VMEM((32, 129), jnp.float32) for _ in range(2)],
          [pltpu.VMEM((32, 128), jnp.float32) for _ in range(2)],
          pltpu.VMEM((16, 128), jnp.float32),
          [pltpu.SemaphoreType.DMA for _ in range(2)],
          [pltpu.SemaphoreType.DMA for _ in range(2)],
          pltpu.SemaphoreType.DMA,
      ],
      compiler_params=pltpu.CompilerParams(needs_layout_passes=False),
  )
  def k(tab_hbm, tail_hbm, out_hbm, in_v, out_v, tout_v, isem, osem, tsem):
    w = _worker_id()

    def blk_of(t):
      return w + t * NUM_WORKERS

    def start_in(t, b):
      pltpu.async_copy(
          tab_hbm.at[:, pl.ds(blk_of(t) * 128, 128)],
          in_v[b].at[:, pl.ds(0, 128)], isem[b])

    def wait_in(b):
      pltpu.make_async_copy(
          tab_hbm.at[:, pl.ds(0, 128)], in_v[b].at[:, pl.ds(0, 128)],
          isem[b]).wait()

    def start_out(t, b):
      pltpu.async_copy(
          out_v[b], out_hbm.at[pl.ds(blk_of(t) * 32, 32)], osem[b])

    def wait_out(b):
      pltpu.make_async_copy(
          out_v[b], out_hbm.at[pl.ds(0, 32)], osem[b]).wait()

    def transpose_block(src, dst, n_plocal):
      for p_local in range(n_plocal):
        for mg in range(8):
          dvec = 16 * (mg & 1) + _iota16()
          lconst = jnp.full((16,), 4 * p_local + (mg >> 1), jnp.int32)
          v = plsc.load_gather(src, [dvec, lconst])
          dst[p_local, pl.ds(16 * mg, 16)] = v

    @pl.when(blk_of(0) < NBLK_FULL)
    def _():
      start_in(0, 0)

    def body(t2, carry):
      for b in range(2):
        t = 2 * t2 + b

        @pl.when(blk_of(t + 1) < NBLK_FULL)
        def _():
          start_in(t + 1, 1 - b)

        @pl.when(blk_of(t) < NBLK_FULL)
        def _():
          wait_in(b)

        @pl.when(jnp.logical_and(t >= 2, blk_of(t) < NBLK_FULL))
        def _():
          wait_out(b)

        @pl.when(blk_of(t) < NBLK_FULL)
        def _():
          transpose_block(in_v[b], out_v[b], 32)
          start_out(t, b)
      return carry

    lax.fori_loop(0, (PACK_ITERS + 1) // 2, body, 0)
    @pl.when(blk_of(1) < NBLK_FULL)
    def _():
      wait_out(0)
      wait_out(1)

    @pl.when(w == 0)
    def _():
      pltpu.async_copy(tail_hbm, tout_v, tsem)
      pltpu.make_async_copy(tail_hbm, tout_v, tsem).wait()
      pltpu.async_copy(tout_v, out_hbm.at[pl.ds(NBLK_FULL * 32, 16)], tsem)
      pltpu.make_async_copy(
          tout_v, out_hbm.at[pl.ds(0, 16)], tsem).wait()

  return k(table_t, tail_pk)


def _gather_packed(packed, idx_t):
  mesh = plsc.VectorSubcoreMesh(core_axis_name="c", subcore_axis_name="s")

  @functools.partial(
      pl.kernel,
      out_type=jax.ShapeDtypeStruct((SEQ, EMBED_DIM, BATCH), jnp.float32),
      mesh=mesh,
      scratch_types=[
          pltpu.VMEM((SEQ, 128 * J_PER_W), jnp.int32),
          [pltpu.VMEM((128, 129), jnp.float32) for _ in range(2)],
          [pltpu.VMEM((1, EMBED_DIM, 128), jnp.float32) for _ in range(2)],
          [pltpu.VMEM((128,), jnp.int32) for _ in range(2)],
          [pltpu.VMEM((128,), jnp.int32) for _ in range(2)],
          [pltpu.SemaphoreType.DMA for _ in range(2)],
          [pltpu.SemaphoreType.DMA for _ in range(2)],
      ],
      compiler_params=pltpu.CompilerParams(needs_layout_passes=False),
  )
  def k(pk_hbm, idx_hbm, out_hbm, idx_v, rows, out_v, pidx, moff, gsem, osem):
    w = _worker_id()
    pltpu.sync_copy(
        idx_hbm.at[:, pl.ds(w * (128 * J_PER_W), 128 * J_PER_W)], idx_v)

    def s_of(u):
      return u >> 2

    def prep(u, b):
      for g in range(8):
        rv = idx_v[s_of(u), pl.ds((u & 3) * 128 + 16 * g, 16)]
        pidx[b][pl.ds(16 * g, 16)] = jnp.right_shift(rv, 2)
        moff[b][pl.ds(16 * g, 16)] = jnp.left_shift(jnp.bitwise_and(rv, 3), 5)
      pltpu.async_copy(pk_hbm.at[pidx[b]], rows[b].at[:, pl.ds(0, 128)],
                       gsem[b])

    def wait_gather(b):
      pltpu.make_async_copy(
          pk_hbm.at[pl.ds(0, 128)], rows[b].at[:, pl.ds(0, 128)],
          gsem[b]).wait()

    def start_out(u, b):
      pltpu.async_copy(
          out_v[b],
          out_hbm.at[pl.ds(s_of(u), 1), pl.ds(0, EMBED_DIM),
                     pl.ds((w * J_PER_W + (u & 3)) * 128, 128)],
          osem[b])

    def wait_out(b):
      pltpu.make_async_copy(
          out_v[b],
          out_hbm.at[pl.ds(0, 1), pl.ds(0, EMBED_DIM), pl.ds(0, 128)],
          osem[b]).wait()

    def extract(b):
      for g in range(8):
        bvec = 16 * g + _iota16()
        mv = moff[b][pl.ds(16 * g, 16)]
        for d in range(EMBED_DIM):
          v = plsc.load_gather(rows[b], [bvec, mv + d])
          out_v[b][0, d, pl.ds(16 * g, 16)] = v

    prep(0, 0)

    def body(t2, carry):
      for b in range(2):
        u = 2 * t2 + b

        @pl.when(u + 1 < UNITS)
        def _():
          prep(u + 1, 1 - b)

        wait_gather(b)

        @pl.when(u >= 2)
        def _():
          wait_out(b)

        extract(b)
        start_out(u, b)
      return carry

    lax.fori_loop(0, UPAIRS, body, 0)
    wait_out(0)
    wait_out(1)

  return k(packed, idx_t)


def kernel(indices, table):
  idx_t = jnp.transpose(indices.astype(jnp.int32), (1, 0))
  packed = jnp.reshape(table.astype(jnp.float32), (PACKED_ROWS, 128))
  out_t = _gather_packed(packed, idx_t)
  return jnp.transpose(out_t, (2, 0, 1))


# speedup vs baseline: 1.2753x; 1.2753x over previous
import functools

import jax
import jax.numpy as jnp
from jax import lax
from jax.experimental import pallas as pl
from jax.experimental.pallas import tpu as pltpu
from jax.experimental.pallas import tpu_sc as plsc

NUM_CORES = 2
NUM_SUBCORES = 16
NUM_WORKERS = NUM_CORES * NUM_SUBCORES

BATCH = 16384
SEQ = 50
EMBED_DIM = 32
VOCAB = 1000000

PACKED_ROWS = VOCAB // 4
NBLK_FULL = VOCAB // 128
TAIL_COLS = VOCAB - NBLK_FULL * 128
PACK_ITERS = (NBLK_FULL + NUM_WORKERS - 1) // NUM_WORKERS

JBLKS = BATCH // 128
J_PER_W = JBLKS // NUM_WORKERS
UNITS = SEQ * J_PER_W
UPAIRS = UNITS // 2


def _worker_id():
  return lax.axis_index("s") * NUM_CORES + lax.axis_index("c")


def _iota16():
  return lax.iota(jnp.int32, 16)


def _pack_table(table_t, tail_pk):
  mesh = plsc.VectorSubcoreMesh(core_axis_name="c", subcore_axis_name="s")

  @functools.partial(
      pl.kernel,
      out_type=jax.ShapeDtypeStruct((PACKED_ROWS, 128), jnp.float32),
      mesh=mesh,
      scratch_types=[
          [pltpu.VMEM((32, 129), jnp.float32) for _ in range(2)],
          [pltpu.VMEM((32, 128), jnp.float32) for _ in range(2)],
          pltpu.VMEM((16, 128), jnp.float32),
          [pltpu.SemaphoreType.DMA for _ in range(2)],
          [pltpu.SemaphoreType.DMA for _ in range(2)],
          pltpu.SemaphoreType.DMA,
      ],
      compiler_params=pltpu.CompilerParams(needs_layout_passes=False),
  )
  def k(tab_hbm, tail_hbm, out_hbm, in_v, out_v, tout_v, isem, osem, tsem):
    w = _worker_id()

    def blk_of(t):
      return w + t * NUM_WORKERS

    def start_in(t, b):
      pltpu.async_copy(
          tab_hbm.at[:, pl.ds(blk_of(t) * 128, 128)],
          in_v[b].at[:, pl.ds(0, 128)], isem[b])

    def wait_in(b):
      pltpu.make_async_copy(
          tab_hbm.at[:, pl.ds(0, 128)], in_v[b].at[:, pl.ds(0, 128)],
          isem[b]).wait()

    def start_out(t, b):
      pltpu.async_copy(
          out_v[b], out_hbm.at[pl.ds(blk_of(t) * 32, 32)], osem[b])

    def wait_out(b):
      pltpu.make_async_copy(
          out_v[b], out_hbm.at[pl.ds(0, 32)], osem[b]).wait()

    def transpose_block(src, dst, n_plocal):
      for p_local in range(n_plocal):
        for mg in range(8):
          dvec = 16 * (mg & 1) + _iota16()
          lconst = jnp.full((16,), 4 * p_local + (mg >> 1), jnp.int32)
          v = plsc.load_gather(src, [dvec, lconst])
          dst[p_local, pl.ds(16 * mg, 16)] = v

    @pl.when(blk_of(0) < NBLK_FULL)
    def _():
      start_in(0, 0)

    def body(t2, carry):
      for b in range(2):
        t = 2 * t2 + b

        @pl.when(blk_of(t + 1) < NBLK_FULL)
        def _():
          start_in(t + 1, 1 - b)

        @pl.when(blk_of(t) < NBLK_FULL)
        def _():
          wait_in(b)

        @pl.when(jnp.logical_and(t >= 2, blk_of(t) < NBLK_FULL))
        def _():
          wait_out(b)

        @pl.when(blk_of(t) < NBLK_FULL)
        def _():
          transpose_block(in_v[b], out_v[b], 32)
          start_out(t, b)
      return carry

    lax.fori_loop(0, (PACK_ITERS + 1) // 2, body, 0)
    @pl.when(blk_of(1) < NBLK_FULL)
    def _():
      wait_out(0)
      wait_out(1)

    @pl.when(w == 0)
    def _():
      pltpu.async_copy(tail_hbm, tout_v, tsem)
      pltpu.make_async_copy(tail_hbm, tout_v, tsem).wait()
      pltpu.async_copy(tout_v, out_hbm.at[pl.ds(NBLK_FULL * 32, 16)], tsem)
      pltpu.make_async_copy(
          tout_v, out_hbm.at[pl.ds(0, 16)], tsem).wait()

  return k(table_t, tail_pk)


def _gather_packed(packed, idx_t):
  mesh = plsc.VectorSubcoreMesh(core_axis_name="c", subcore_axis_name="s")

  @functools.partial(
      pl.kernel,
      out_type=jax.ShapeDtypeStruct((SEQ, EMBED_DIM, BATCH), jnp.float32),
      mesh=mesh,
      scratch_types=[
          pltpu.VMEM((SEQ, 128 * J_PER_W), jnp.int32),
          [pltpu.VMEM((128, 128), jnp.float32) for _ in range(2)],
          [pltpu.VMEM((1, EMBED_DIM, 128), jnp.float32) for _ in range(2)],
          [pltpu.VMEM((128,), jnp.int32) for _ in range(2)],
          [pltpu.VMEM((128,), jnp.int32) for _ in range(2)],
          [pltpu.SemaphoreType.DMA for _ in range(2)],
          [pltpu.SemaphoreType.DMA for _ in range(2)],
      ],
      compiler_params=pltpu.CompilerParams(needs_layout_passes=False),
  )
  def k(pk_hbm, idx_hbm, out_hbm, idx_v, rows, out_v, pidx, moff, gsem, osem):
    w = _worker_id()
    pltpu.sync_copy(
        idx_hbm.at[:, pl.ds(w * (128 * J_PER_W), 128 * J_PER_W)], idx_v)

    def s_of(u):
      return u >> 2

    def prep(u, b):
      for g in range(8):
        rv = idx_v[s_of(u), pl.ds((u & 3) * 128 + 16 * g, 16)]
        pidx[b][pl.ds(16 * g, 16)] = jnp.right_shift(rv, 2)
        moff[b][pl.ds(16 * g, 16)] = jnp.left_shift(jnp.bitwise_and(rv, 3), 5)
      pltpu.async_copy(pk_hbm.at[pidx[b]], rows[b], gsem[b])

    def wait_gather(b):
      pltpu.make_async_copy(
          pk_hbm.at[pl.ds(0, 128)], rows[b], gsem[b]).wait()

    def start_out(u, b):
      pltpu.async_copy(
          out_v[b],
          out_hbm.at[pl.ds(s_of(u), 1), pl.ds(0, EMBED_DIM),
                     pl.ds((w * J_PER_W + (u & 3)) * 128, 128)],
          osem[b])

    def wait_out(b):
      pltpu.make_async_copy(
          out_v[b],
          out_hbm.at[pl.ds(0, 1), pl.ds(0, EMBED_DIM), pl.ds(0, 128)],
          osem[b]).wait()

    z16 = jnp.zeros((16,), jnp.int32)
    dvs = tuple(jnp.bitwise_and(d0 + _iota16(), EMBED_DIM - 1)
                for d0 in range(EMBED_DIM))
    bvs = tuple(16 * g + _iota16() for g in range(8))

    def extract(b):
      for g in range(8):
        mv = moff[b][pl.ds(16 * g, 16)]
        for d0 in range(EMBED_DIM):
          v = plsc.load_gather(rows[b], [bvs[g], mv + dvs[d0]])
          plsc.store_scatter(out_v[b], [z16, dvs[d0], bvs[g]], v)

    prep(0, 0)

    def body(t2, carry):
      for b in range(2):
        u = 2 * t2 + b

        @pl.when(u + 1 < UNITS)
        def _():
          prep(u + 1, 1 - b)

        wait_gather(b)

        @pl.when(u >= 2)
        def _():
          wait_out(b)

        extract(b)
        start_out(u, b)
      return carry

    lax.fori_loop(0, UPAIRS, body, 0)
    wait_out(0)
    wait_out(1)

  return k(packed, idx_t)


def kernel(indices, table):
  idx_t = jnp.transpose(indices.astype(jnp.int32), (1, 0))
  packed = jnp.reshape(table.astype(jnp.float32), (PACKED_ROWS, 128))
  out_t = _gather_packed(packed, idx_t)
  return jnp.transpose(out_t, (2, 0, 1))
